# baseline (speedup 1.0000x reference)
import jax
import jax.numpy as jnp
from jax.experimental import pallas as pl
from jax.experimental.pallas import tpu as pltpu

_NUM_CLASSES = 30522
_ROWS = 1024
_BC = 512


def _onehot_block(x_ref, o_ref):
    j = pl.program_id(0)
    cols = jax.lax.broadcasted_iota(jnp.int32, (_ROWS, _BC), 1) + j * _BC
    o_ref[...] = (x_ref[...] == cols).astype(jnp.int32)


def kernel(x):
    x2 = x.reshape(_ROWS, 1)
    grid = (pl.cdiv(_NUM_CLASSES, _BC),)
    return pl.pallas_call(
        _onehot_block,
        grid=grid,
        in_specs=[pl.BlockSpec((_ROWS, 1), lambda j: (0, 0))],
        out_specs=pl.BlockSpec((_ROWS, _BC), lambda j: (0, j)),
        out_shape=jax.ShapeDtypeStruct((_ROWS, _NUM_CLASSES), jnp.int32),
        compiler_params=pltpu.CompilerParams(
            dimension_semantics=("arbitrary",),
        ),
    )(x2)


# speedup vs baseline: 1.1067x; 1.1067x over previous
import jax
import jax.numpy as jnp
from jax.experimental import pallas as pl
from jax.experimental.pallas import tpu as pltpu

_NUM_CLASSES = 30522
_ROWS = 1024
_BR = 32


def _onehot_block(x_ref, o_ref):
    cols = jax.lax.broadcasted_iota(jnp.int32, (_BR, _NUM_CLASSES), 1)
    o_ref[...] = (x_ref[...] == cols).astype(jnp.int32)


def kernel(x):
    x2 = x.reshape(_ROWS, 1)
    grid = (_ROWS // _BR,)
    return pl.pallas_call(
        _onehot_block,
        grid=grid,
        in_specs=[pl.BlockSpec((_BR, 1), lambda i: (i, 0))],
        out_specs=pl.BlockSpec((_BR, _NUM_CLASSES), lambda i: (i, 0)),
        out_shape=jax.ShapeDtypeStruct((_ROWS, _NUM_CLASSES), jnp.int32),
        compiler_params=pltpu.CompilerParams(
            dimension_semantics=("parallel",),
        ),
    )(x2)
